# Initial kernel scaffold; baseline (speedup 1.0000x reference)
import jax
import jax.numpy as jnp
from jax.experimental import pallas as pl


def kernel(x, edge_index, W_self, W_neigh, b):
    raise NotImplementedError("write your pallas kernel here")


# speedup vs baseline: 3.9108x; 3.9108x over previous
import functools

import jax
import jax.numpy as jnp
from jax import lax
from jax.experimental import pallas as pl
from jax.experimental.pallas import tpu as pltpu
from jax.experimental.pallas import tpu_sc as plsc


def _sc_segment_sum(xP, srcs2, dst, zeros, *, N, NP, E, HP, K):
    info = plsc.get_sparse_core_info()
    NC, NS = info.num_cores, info.num_subcores
    epw = E // NS
    nblk = epw // K
    tail = epw - nblk * K
    rpw = NP // NS

    mesh = plsc.VectorSubcoreMesh(core_axis_name="c", subcore_axis_name="s")

    @functools.partial(
        pl.kernel,
        out_type=jax.ShapeDtypeStruct((NC * NP, HP), jnp.float32),
        mesh=mesh,
        scratch_types=[
            pltpu.VMEM((K,), jnp.int32),
            pltpu.VMEM((K,), jnp.int32),
            pltpu.VMEM((K, HP), jnp.float32),
            pltpu.VMEM((max(tail, 8),), jnp.int32),
            pltpu.VMEM((max(tail, 8),), jnp.int32),
            pltpu.VMEM((max(tail, 8), HP), jnp.float32),
            pltpu.VMEM_SHARED((NP, HP), jnp.float32),
            pltpu.SemaphoreType.DMA,
        ],
        compiler_params=pltpu.CompilerParams(use_tc_tiling_on_sc=False),
    )
    def sc_agg(xp_hbm, src_hbm, dst_hbm, z_hbm, out_hbm,
               si, di, rows, si8, di8, rows8, acc, sem):
        c = lax.axis_index("c")
        s = lax.axis_index("s")
        pltpu.sync_copy(z_hbm.at[pl.ds(s * rpw, rpw)],
                        acc.at[pl.ds(s * rpw, rpw)])
        plsc.subcore_barrier()

        base0 = s * epw

        def body(j, carry):
            base = base0 + j * K
            pltpu.sync_copy(src_hbm.at[pl.ds(c * E + base, K)], si)
            pltpu.sync_copy(dst_hbm.at[pl.ds(base, K)], di)
            pltpu.async_copy(xp_hbm.at[si], rows, sem).wait()
            pltpu.async_copy(rows, acc.at[di], sem, add=True).wait()
            return carry

        lax.fori_loop(0, nblk, body, 0, unroll=False)

        if tail:
            tb = base0 + nblk * K
            pltpu.sync_copy(src_hbm.at[pl.ds(c * E + tb, tail)], si8)
            pltpu.sync_copy(dst_hbm.at[pl.ds(tb, tail)], di8)
            pltpu.async_copy(xp_hbm.at[si8], rows8, sem).wait()
            pltpu.async_copy(rows8, acc.at[di8], sem, add=True).wait()

        plsc.subcore_barrier()
        pltpu.sync_copy(acc.at[pl.ds(s * rpw, rpw)],
                        out_hbm.at[pl.ds(c * NP + s * rpw, rpw)])

    return sc_agg(xP, srcs2, dst, zeros)


def _tc_dense(x, aggL, aggR, cnt, W_self, WnT, WnB, b2, *, N, D, B):
    H = D // 2

    def body(x_ref, al_ref, ar_ref, cnt_ref, ws_ref, wt_ref, wb_ref, b_ref,
             out_ref):
        r = 1.0 / jnp.maximum(cnt_ref[...], 1.0)
        dn = (((1,), (0,)), ((), ()))
        acc = lax.dot_general(x_ref[...], ws_ref[...], dn,
                              precision=lax.Precision.HIGHEST,
                              preferred_element_type=jnp.float32)
        acc += lax.dot_general(al_ref[...] * r, wt_ref[...], dn,
                               precision=lax.Precision.HIGHEST,
                               preferred_element_type=jnp.float32)
        acc += lax.dot_general(ar_ref[...] * r, wb_ref[...], dn,
                               precision=lax.Precision.HIGHEST,
                               preferred_element_type=jnp.float32)
        out_ref[...] = acc + b_ref[...]

    grid = (N // B,)
    return pl.pallas_call(
        body,
        grid=grid,
        in_specs=[
            pl.BlockSpec((B, D), lambda i: (i, 0)),
            pl.BlockSpec((B, H), lambda i: (i, 0)),
            pl.BlockSpec((B, H), lambda i: (i, 0)),
            pl.BlockSpec((B, 1), lambda i: (i, 0)),
            pl.BlockSpec((D, D), lambda i: (0, 0)),
            pl.BlockSpec((H, D), lambda i: (0, 0)),
            pl.BlockSpec((H, D), lambda i: (0, 0)),
            pl.BlockSpec((1, D), lambda i: (0, 0)),
        ],
        out_specs=pl.BlockSpec((B, D), lambda i: (i, 0)),
        out_shape=jax.ShapeDtypeStruct((N, D), jnp.float32),
    )(x, aggL, aggR, cnt, W_self, WnT, WnB, b2)


def kernel(x, edge_index, W_self, W_neigh, b):
    N, D = x.shape
    E = edge_index.shape[1]
    H = D // 2
    PAD = 16
    HP = H + PAD
    K = 128
    NP = ((N + 127) // 128) * 128

    ones = jnp.ones((N, PAD), jnp.float32)
    xP = jnp.concatenate(
        [jnp.concatenate([x[:, :H], ones], axis=1),
         jnp.concatenate([x[:, H:], ones], axis=1)], axis=0)
    src = edge_index[0]
    dst = edge_index[1]
    srcs2 = jnp.concatenate([src, src + N])
    zeros = jnp.zeros((NP, HP), jnp.float32)

    agg2 = _sc_segment_sum(xP, srcs2, dst, zeros, N=N, NP=NP, E=E, HP=HP,
                           K=K)

    aggL = agg2[:N, :H]
    aggR = agg2[NP:NP + N, :H]
    cnt = agg2[:N, H:H + 1]
    WnT = W_neigh[:H, :]
    WnB = W_neigh[H:, :]
    b2 = b.reshape(1, D)

    return _tc_dense(x, aggL, aggR, cnt, W_self, WnT, WnB, b2,
                     N=N, D=D, B=2000)


# speedup vs baseline: 5.6564x; 1.4464x over previous
import functools

import jax
import jax.numpy as jnp
from jax import lax
from jax.experimental import pallas as pl
from jax.experimental.pallas import tpu as pltpu
from jax.experimental.pallas import tpu_sc as plsc


def _sc_segment_sum(xP, srcs2, dst2, zeros, *, N, E, HP, K):
    info = plsc.get_sparse_core_info()
    NC, NS = info.num_cores, info.num_subcores
    NB = E // K
    bpw = NB // NS
    nx = NB - bpw * NS
    rA = ((N + NS - 1) // NS + 7) // 8 * 8
    rB = N - rA * (NS - 1)
    assert bpw % 2 == 0 and rB > 0 and rB % 8 == 0

    mesh = plsc.VectorSubcoreMesh(core_axis_name="c", subcore_axis_name="s")

    @functools.partial(
        pl.kernel,
        out_type=jax.ShapeDtypeStruct((NC * N, HP), jnp.float32),
        mesh=mesh,
        scratch_types=[
            pltpu.VMEM((bpw + 1, K), jnp.int32),
            pltpu.VMEM((bpw + 1, K), jnp.int32),
            pltpu.VMEM((K, HP), jnp.float32),
            pltpu.VMEM((K, HP), jnp.float32),
            pltpu.VMEM_SHARED((N, HP), jnp.float32),
            pltpu.SemaphoreType.DMA,
            pltpu.SemaphoreType.DMA,
            pltpu.SemaphoreType.DMA,
            pltpu.SemaphoreType.DMA,
        ],
        compiler_params=pltpu.CompilerParams(use_tc_tiling_on_sc=False),
    )
    def sc_agg(xp_hbm, src_hbm, dst_hbm, z_hbm, out_hbm,
               si, di, rows0, rows1, acc, gsem0, gsem1, ssem0, ssem1):
        c = lax.axis_index("c")
        s = lax.axis_index("s")

        @pl.when(s < NS - 1)
        def _():
            pltpu.sync_copy(z_hbm.at[pl.ds(s * rA, rA)],
                            acc.at[pl.ds(s * rA, rA)])

        @pl.when(s == NS - 1)
        def _():
            pltpu.sync_copy(z_hbm.at[pl.ds((NS - 1) * rA, rB)],
                            acc.at[pl.ds((NS - 1) * rA, rB)])

        b0 = s * bpw
        pltpu.sync_copy(src_hbm.at[pl.ds(c * NB + b0, bpw)],
                        si.at[pl.ds(0, bpw)])
        pltpu.sync_copy(dst_hbm.at[pl.ds(b0, bpw)], di.at[pl.ds(0, bpw)])

        @pl.when(s < nx)
        def _():
            xb = NB - nx + s
            pltpu.sync_copy(src_hbm.at[pl.ds(c * NB + xb, 1)],
                            si.at[pl.ds(bpw, 1)])
            pltpu.sync_copy(dst_hbm.at[pl.ds(xb, 1)], di.at[pl.ds(bpw, 1)])

        plsc.subcore_barrier()

        rows = (rows0, rows1)
        gsem = (gsem0, gsem1)
        ssem = (ssem0, ssem1)

        pltpu.async_copy(xp_hbm.at[si.at[0]], rows0, gsem0)

        def step(g, j, k):
            @pl.when(j >= 1)
            def _():
                pltpu.make_async_copy(
                    rows[1 - k], acc.at[di.at[0]], ssem[1 - k]).wait()

            @pl.when(j + 1 < bpw)
            def _():
                pltpu.async_copy(xp_hbm.at[si.at[j + 1]], rows[1 - k],
                                 gsem[1 - k])

            pltpu.make_async_copy(xp_hbm.at[si.at[0]], rows[k],
                                  gsem[k]).wait()
            pltpu.async_copy(rows[k], acc.at[di.at[j]], ssem[k], add=True)

        def body(g, carry):
            step(g, 2 * g, 0)
            step(g, 2 * g + 1, 1)
            return carry

        lax.fori_loop(0, bpw // 2, body, 0, unroll=False)

        pltpu.make_async_copy(rows1, acc.at[di.at[0]], ssem1).wait()

        @pl.when(s < nx)
        def _():
            pltpu.async_copy(xp_hbm.at[si.at[bpw]], rows0, gsem0).wait()
            pltpu.async_copy(rows0, acc.at[di.at[bpw]], ssem0,
                             add=True).wait()

        plsc.subcore_barrier()

        @pl.when(s < NS - 1)
        def _():
            pltpu.sync_copy(acc.at[pl.ds(s * rA, rA)],
                            out_hbm.at[pl.ds(c * N + s * rA, rA)])

        @pl.when(s == NS - 1)
        def _():
            pltpu.sync_copy(acc.at[pl.ds((NS - 1) * rA, rB)],
                            out_hbm.at[pl.ds(c * N + (NS - 1) * rA, rB)])

    return sc_agg(xP, srcs2, dst2, zeros)


def _tc_dense(x, aggL, aggR, cnt, W_self, WnT, WnB, b2, *, N, D, B):
    H = D // 2

    def body(x_ref, al_ref, ar_ref, cnt_ref, ws_ref, wt_ref, wb_ref, b_ref,
             out_ref):
        r = 1.0 / jnp.maximum(cnt_ref[...], 1.0)
        dn = (((1,), (0,)), ((), ()))
        acc = lax.dot_general(x_ref[...], ws_ref[...], dn,
                              precision=lax.Precision.HIGHEST,
                              preferred_element_type=jnp.float32)
        acc += lax.dot_general(al_ref[...] * r, wt_ref[...], dn,
                               precision=lax.Precision.HIGHEST,
                               preferred_element_type=jnp.float32)
        acc += lax.dot_general(ar_ref[...] * r, wb_ref[...], dn,
                               precision=lax.Precision.HIGHEST,
                               preferred_element_type=jnp.float32)
        out_ref[...] = acc + b_ref[...]

    grid = (N // B,)
    return pl.pallas_call(
        body,
        grid=grid,
        in_specs=[
            pl.BlockSpec((B, D), lambda i: (i, 0)),
            pl.BlockSpec((B, H), lambda i: (i, 0)),
            pl.BlockSpec((B, H), lambda i: (i, 0)),
            pl.BlockSpec((B, 1), lambda i: (i, 0)),
            pl.BlockSpec((D, D), lambda i: (0, 0)),
            pl.BlockSpec((H, D), lambda i: (0, 0)),
            pl.BlockSpec((H, D), lambda i: (0, 0)),
            pl.BlockSpec((1, D), lambda i: (0, 0)),
        ],
        out_specs=pl.BlockSpec((B, D), lambda i: (i, 0)),
        out_shape=jax.ShapeDtypeStruct((N, D), jnp.float32),
    )(x, aggL, aggR, cnt, W_self, WnT, WnB, b2)


def kernel(x, edge_index, W_self, W_neigh, b):
    N, D = x.shape
    E = edge_index.shape[1]
    H = D // 2
    PAD = 16
    HP = H + PAD
    K = 64

    ones = jnp.ones((N, PAD), jnp.float32)
    xP = jnp.concatenate(
        [jnp.concatenate([x[:, :H], ones], axis=1),
         jnp.concatenate([x[:, H:], ones], axis=1)], axis=0)
    src = edge_index[0]
    dst = edge_index[1]
    srcs2 = jnp.concatenate([src, src + N]).reshape(2 * E // K, K)
    dst2 = dst.reshape(E // K, K)
    zeros = jnp.zeros((N, HP), jnp.float32)

    agg2 = _sc_segment_sum(xP, srcs2, dst2, zeros, N=N, E=E, HP=HP, K=K)

    aggL = agg2[:N, :H]
    aggR = agg2[N:, :H]
    cnt = agg2[:N, H:H + 1]
    WnT = W_neigh[:H, :]
    WnB = W_neigh[H:, :]
    b2 = b.reshape(1, D)

    return _tc_dense(x, aggL, aggR, cnt, W_self, WnT, WnB, b2,
                     N=N, D=D, B=2000)


# speedup vs baseline: 6.0632x; 1.0719x over previous
import functools

import jax
import jax.numpy as jnp
from jax import lax
from jax.experimental import pallas as pl
from jax.experimental.pallas import tpu as pltpu
from jax.experimental.pallas import tpu_sc as plsc


def _sc_segment_sum(xP, srcs2, dst2, zeros, *, N, E, HP, K):
    info = plsc.get_sparse_core_info()
    NC, NS = info.num_cores, info.num_subcores
    NB = E // K
    bpw = NB // NS
    nx = NB - bpw * NS
    rA = ((N + NS - 1) // NS + 7) // 8 * 8
    rB = N - rA * (NS - 1)
    assert bpw % 2 == 0 and rB > 0 and rB % 8 == 0

    mesh = plsc.VectorSubcoreMesh(core_axis_name="c", subcore_axis_name="s")

    @functools.partial(
        pl.kernel,
        out_type=jax.ShapeDtypeStruct((NC * N, HP), jnp.float32),
        mesh=mesh,
        scratch_types=[
            pltpu.VMEM((bpw + 1, K), jnp.int32),
            pltpu.VMEM((bpw + 1, K), jnp.int32),
            pltpu.VMEM((K, HP), jnp.float32),
            pltpu.VMEM((K, HP), jnp.float32),
            pltpu.VMEM_SHARED((N, HP), jnp.float32),
            pltpu.SemaphoreType.DMA,
            pltpu.SemaphoreType.DMA,
            pltpu.SemaphoreType.DMA,
            pltpu.SemaphoreType.DMA,
        ],
        compiler_params=pltpu.CompilerParams(use_tc_tiling_on_sc=False),
    )
    def sc_agg(xp_hbm, src_hbm, dst_hbm, z_hbm, out_hbm,
               si, di, rows0, rows1, acc, gsem0, gsem1, ssem0, ssem1):
        c = lax.axis_index("c")
        s = lax.axis_index("s")

        @pl.when(s < NS - 1)
        def _():
            pltpu.sync_copy(z_hbm.at[pl.ds(s * rA, rA)],
                            acc.at[pl.ds(s * rA, rA)])

        @pl.when(s == NS - 1)
        def _():
            pltpu.sync_copy(z_hbm.at[pl.ds((NS - 1) * rA, rB)],
                            acc.at[pl.ds((NS - 1) * rA, rB)])

        b0 = s * bpw
        pltpu.sync_copy(src_hbm.at[pl.ds(c * NB + b0, bpw)],
                        si.at[pl.ds(0, bpw)])
        pltpu.sync_copy(dst_hbm.at[pl.ds(b0, bpw)], di.at[pl.ds(0, bpw)])

        @pl.when(s < nx)
        def _():
            xb = NB - nx + s
            pltpu.sync_copy(src_hbm.at[pl.ds(c * NB + xb, 1)],
                            si.at[pl.ds(bpw, 1)])
            pltpu.sync_copy(dst_hbm.at[pl.ds(xb, 1)], di.at[pl.ds(bpw, 1)])

        plsc.subcore_barrier()

        rows = (rows0, rows1)
        gsem = (gsem0, gsem1)
        ssem = (ssem0, ssem1)

        pltpu.async_copy(xp_hbm.at[si.at[0]], rows0, gsem0)

        def step(g, j, k):
            @pl.when(j >= 1)
            def _():
                pltpu.make_async_copy(
                    rows[1 - k], acc.at[di.at[0]], ssem[1 - k]).wait()

            @pl.when(j + 1 < bpw)
            def _():
                pltpu.async_copy(xp_hbm.at[si.at[j + 1]], rows[1 - k],
                                 gsem[1 - k])

            pltpu.make_async_copy(xp_hbm.at[si.at[0]], rows[k],
                                  gsem[k]).wait()
            pltpu.async_copy(rows[k], acc.at[di.at[j]], ssem[k], add=True)

        def body(g, carry):
            step(g, 2 * g, 0)
            step(g, 2 * g + 1, 1)
            return carry

        lax.fori_loop(0, bpw // 2, body, 0, unroll=False)

        pltpu.make_async_copy(rows1, acc.at[di.at[0]], ssem1).wait()

        @pl.when(s < nx)
        def _():
            pltpu.async_copy(xp_hbm.at[si.at[bpw]], rows0, gsem0).wait()
            pltpu.async_copy(rows0, acc.at[di.at[bpw]], ssem0,
                             add=True).wait()

        plsc.subcore_barrier()

        @pl.when(s < NS - 1)
        def _():
            pltpu.sync_copy(acc.at[pl.ds(s * rA, rA)],
                            out_hbm.at[pl.ds(c * N + s * rA, rA)])

        @pl.when(s == NS - 1)
        def _():
            pltpu.sync_copy(acc.at[pl.ds((NS - 1) * rA, rB)],
                            out_hbm.at[pl.ds(c * N + (NS - 1) * rA, rB)])

    return sc_agg(xP, srcs2, dst2, zeros)


def _tc_self(x, W_self, b2, *, N, D, B):
    def body(x_ref, ws_ref, b_ref, out_ref):
        dn = (((1,), (0,)), ((), ()))
        out_ref[...] = lax.dot_general(
            x_ref[...], ws_ref[...], dn,
            precision=lax.Precision.HIGHEST,
            preferred_element_type=jnp.float32) + b_ref[...]

    return pl.pallas_call(
        body,
        grid=(N // B,),
        in_specs=[
            pl.BlockSpec((B, D), lambda i: (i, 0)),
            pl.BlockSpec((D, D), lambda i: (0, 0)),
            pl.BlockSpec((1, D), lambda i: (0, 0)),
        ],
        out_specs=pl.BlockSpec((B, D), lambda i: (i, 0)),
        out_shape=jax.ShapeDtypeStruct((N, D), jnp.float32),
    )(x, W_self, b2)


def _tc_neigh(self_out, agg2, WnT, WnB, *, N, D, HP, B):
    H = D // 2

    def body(s_ref, al_ref, ar_ref, wt_ref, wb_ref, out_ref):
        r = 1.0 / jnp.maximum(al_ref[:, H:H + 1], 1.0)
        dn = (((1,), (0,)), ((), ()))
        acc = lax.dot_general(al_ref[:, :H] * r, wt_ref[...], dn,
                              precision=lax.Precision.HIGHEST,
                              preferred_element_type=jnp.float32)
        acc += lax.dot_general(ar_ref[:, :H] * r, wb_ref[...], dn,
                               precision=lax.Precision.HIGHEST,
                               preferred_element_type=jnp.float32)
        out_ref[...] = acc + s_ref[...]

    nb = N // B
    return pl.pallas_call(
        body,
        grid=(nb,),
        in_specs=[
            pl.BlockSpec((B, D), lambda i: (i, 0)),
            pl.BlockSpec((B, HP), lambda i: (i, 0)),
            pl.BlockSpec((B, HP), lambda i: (nb + i, 0)),
            pl.BlockSpec((H, D), lambda i: (0, 0)),
            pl.BlockSpec((H, D), lambda i: (0, 0)),
        ],
        out_specs=pl.BlockSpec((B, D), lambda i: (i, 0)),
        out_shape=jax.ShapeDtypeStruct((N, D), jnp.float32),
    )(self_out, agg2, agg2, WnT, WnB)


def kernel(x, edge_index, W_self, W_neigh, b):
    N, D = x.shape
    E = edge_index.shape[1]
    H = D // 2
    PAD = 16
    HP = H + PAD
    K = 64

    ones = jnp.ones((N, PAD), jnp.float32)
    xP = jnp.concatenate(
        [jnp.concatenate([x[:, :H], ones], axis=1),
         jnp.concatenate([x[:, H:], ones], axis=1)], axis=0)
    src = edge_index[0]
    dst = edge_index[1]
    srcs2 = jnp.concatenate([src, src + N]).reshape(2 * E // K, K)
    dst2 = dst.reshape(E // K, K)
    zeros = jnp.zeros((N, HP), jnp.float32)

    agg2 = _sc_segment_sum(xP, srcs2, dst2, zeros, N=N, E=E, HP=HP, K=K)

    WnT = W_neigh[:H, :]
    WnB = W_neigh[H:, :]
    b2 = b.reshape(1, D)

    self_out = _tc_self(x, W_self, b2, N=N, D=D, B=2000)
    return _tc_neigh(self_out, agg2, WnT, WnB, N=N, D=D, HP=HP, B=2000)


# speedup vs baseline: 7.3212x; 1.2075x over previous
import functools

import jax
import jax.numpy as jnp
from jax import lax
from jax.experimental import pallas as pl
from jax.experimental.pallas import tpu as pltpu
from jax.experimental.pallas import tpu_sc as plsc


def _sc_segment_sum(xP, srcs2, dst2, zA, zC, onesK, *, N, E, H, K):
    info = plsc.get_sparse_core_info()
    NC, NS = info.num_cores, info.num_subcores
    NB = E // K
    bpw = NB // NS
    nx = NB - bpw * NS
    rA = ((N + NS - 1) // NS + 7) // 8 * 8
    rB = N - rA * (NS - 1)
    assert bpw % 2 == 0 and rB > 0 and rB % 8 == 0

    mesh = plsc.VectorSubcoreMesh(core_axis_name="c", subcore_axis_name="s")

    @functools.partial(
        pl.kernel,
        out_type=(jax.ShapeDtypeStruct((NC * N, H), jnp.float32),
                  jax.ShapeDtypeStruct((N, 16), jnp.float32)),
        mesh=mesh,
        scratch_types=[
            pltpu.VMEM((bpw + 1, K), jnp.int32),
            pltpu.VMEM((bpw + 1, K), jnp.int32),
            pltpu.VMEM((K, H), jnp.float32),
            pltpu.VMEM((K, H), jnp.float32),
            pltpu.VMEM((K, 16), jnp.float32),
            pltpu.VMEM_SHARED((N, H), jnp.float32),
            pltpu.VMEM_SHARED((N, 16), jnp.float32),
            pltpu.SemaphoreType.DMA,
            pltpu.SemaphoreType.DMA,
            pltpu.SemaphoreType.DMA,
            pltpu.SemaphoreType.DMA,
            pltpu.SemaphoreType.DMA,
        ],
        compiler_params=pltpu.CompilerParams(use_tc_tiling_on_sc=False),
    )
    def sc_agg(xp_hbm, src_hbm, dst_hbm, za_hbm, zc_hbm, ones_hbm,
               out_hbm, cnt_hbm,
               si, di, rows0, rows1, onesv, acc, cnt,
               gsem0, gsem1, ssem0, ssem1, csem):
        c = lax.axis_index("c")
        s = lax.axis_index("s")

        @pl.when(s < NS - 1)
        def _():
            pltpu.sync_copy(za_hbm.at[pl.ds(0, rA)],
                            acc.at[pl.ds(s * rA, rA)])

        @pl.when(s == NS - 1)
        def _():
            pltpu.sync_copy(za_hbm.at[pl.ds(0, rB)],
                            acc.at[pl.ds((NS - 1) * rA, rB)])

        @pl.when((c == 0) & (s < NS - 1))
        def _():
            pltpu.sync_copy(zc_hbm.at[pl.ds(0, rA)],
                            cnt.at[pl.ds(s * rA, rA)])

        @pl.when((c == 0) & (s == NS - 1))
        def _():
            pltpu.sync_copy(zc_hbm.at[pl.ds(0, rB)],
                            cnt.at[pl.ds((NS - 1) * rA, rB)])

        @pl.when(c == 0)
        def _():
            pltpu.sync_copy(ones_hbm, onesv)

        b0 = s * bpw
        pltpu.sync_copy(src_hbm.at[pl.ds(c * NB + b0, bpw)],
                        si.at[pl.ds(0, bpw)])
        pltpu.sync_copy(dst_hbm.at[pl.ds(b0, bpw)], di.at[pl.ds(0, bpw)])

        @pl.when(s < nx)
        def _():
            xb = NB - nx + s
            pltpu.sync_copy(src_hbm.at[pl.ds(c * NB + xb, 1)],
                            si.at[pl.ds(bpw, 1)])
            pltpu.sync_copy(dst_hbm.at[pl.ds(xb, 1)], di.at[pl.ds(bpw, 1)])

        plsc.subcore_barrier()

        rows = (rows0, rows1)
        gsem = (gsem0, gsem1)
        ssem = (ssem0, ssem1)

        pltpu.async_copy(xp_hbm.at[si.at[0]], rows0, gsem0)

        def step(j, k):
            @pl.when(j >= 1)
            def _():
                pltpu.make_async_copy(
                    rows[1 - k], acc.at[di.at[0]], ssem[1 - k]).wait()

            @pl.when(j + 1 < bpw)
            def _():
                pltpu.async_copy(xp_hbm.at[si.at[j + 1]], rows[1 - k],
                                 gsem[1 - k])

            pltpu.make_async_copy(xp_hbm.at[si.at[0]], rows[k],
                                  gsem[k]).wait()
            pltpu.async_copy(rows[k], acc.at[di.at[j]], ssem[k], add=True)

            @pl.when((c == 0) & (j >= 1))
            def _():
                pltpu.make_async_copy(onesv, cnt.at[di.at[0]], csem).wait()

            @pl.when(c == 0)
            def _():
                pltpu.async_copy(onesv, cnt.at[di.at[j]], csem, add=True)

        def body(g, carry):
            step(2 * g, 0)
            step(2 * g + 1, 1)
            return carry

        lax.fori_loop(0, bpw // 2, body, 0, unroll=False)

        pltpu.make_async_copy(rows1, acc.at[di.at[0]], ssem1).wait()

        @pl.when(c == 0)
        def _():
            pltpu.make_async_copy(onesv, cnt.at[di.at[0]], csem).wait()

        @pl.when(s < nx)
        def _():
            pltpu.async_copy(xp_hbm.at[si.at[bpw]], rows0, gsem0).wait()
            pltpu.async_copy(rows0, acc.at[di.at[bpw]], ssem0,
                             add=True).wait()

            @pl.when(c == 0)
            def _():
                pltpu.async_copy(onesv, cnt.at[di.at[bpw]], csem,
                                 add=True).wait()

        plsc.subcore_barrier()

        @pl.when(s < NS - 1)
        def _():
            pltpu.sync_copy(acc.at[pl.ds(s * rA, rA)],
                            out_hbm.at[pl.ds(c * N + s * rA, rA)])

        @pl.when(s == NS - 1)
        def _():
            pltpu.sync_copy(acc.at[pl.ds((NS - 1) * rA, rB)],
                            out_hbm.at[pl.ds(c * N + (NS - 1) * rA, rB)])

        @pl.when((c == 0) & (s < NS - 1))
        def _():
            pltpu.sync_copy(cnt.at[pl.ds(s * rA, rA)],
                            cnt_hbm.at[pl.ds(s * rA, rA)])

        @pl.when((c == 0) & (s == NS - 1))
        def _():
            pltpu.sync_copy(cnt.at[pl.ds((NS - 1) * rA, rB)],
                            cnt_hbm.at[pl.ds((NS - 1) * rA, rB)])

    return sc_agg(xP, srcs2, dst2, zA, zC, onesK)


def _tc_self(x, W_self, b2, *, N, D, B):
    def body(x_ref, ws_ref, b_ref, out_ref):
        dn = (((1,), (0,)), ((), ()))
        out_ref[...] = lax.dot_general(
            x_ref[...], ws_ref[...], dn,
            precision=lax.Precision.HIGHEST,
            preferred_element_type=jnp.float32) + b_ref[...]

    return pl.pallas_call(
        body,
        grid=(N // B,),
        in_specs=[
            pl.BlockSpec((B, D), lambda i: (i, 0)),
            pl.BlockSpec((D, D), lambda i: (0, 0)),
            pl.BlockSpec((1, D), lambda i: (0, 0)),
        ],
        out_specs=pl.BlockSpec((B, D), lambda i: (i, 0)),
        out_shape=jax.ShapeDtypeStruct((N, D), jnp.float32),
    )(x, W_self, b2)


def _tc_neigh(self_out, agg2, cnt, WnT, WnB, *, N, D, B):
    H = D // 2

    def body(s_ref, al_ref, ar_ref, cnt_ref, wt_ref, wb_ref, out_ref):
        r = 1.0 / jnp.maximum(cnt_ref[:, :1], 1.0)
        dn = (((1,), (0,)), ((), ()))
        acc = lax.dot_general(al_ref[...] * r, wt_ref[...], dn,
                              precision=lax.Precision.HIGHEST,
                              preferred_element_type=jnp.float32)
        acc += lax.dot_general(ar_ref[...] * r, wb_ref[...], dn,
                               precision=lax.Precision.HIGHEST,
                               preferred_element_type=jnp.float32)
        out_ref[...] = acc + s_ref[...]

    nb = N // B
    return pl.pallas_call(
        body,
        grid=(nb,),
        in_specs=[
            pl.BlockSpec((B, D), lambda i: (i, 0)),
            pl.BlockSpec((B, H), lambda i: (i, 0)),
            pl.BlockSpec((B, H), lambda i: (nb + i, 0)),
            pl.BlockSpec((B, 16), lambda i: (i, 0)),
            pl.BlockSpec((H, D), lambda i: (0, 0)),
            pl.BlockSpec((H, D), lambda i: (0, 0)),
        ],
        out_specs=pl.BlockSpec((B, D), lambda i: (i, 0)),
        out_shape=jax.ShapeDtypeStruct((N, D), jnp.float32),
    )(self_out, agg2, agg2, cnt, WnT, WnB)


def kernel(x, edge_index, W_self, W_neigh, b):
    N, D = x.shape
    E = edge_index.shape[1]
    H = D // 2
    K = 64
    NS = 16
    rA = ((N + NS - 1) // NS + 7) // 8 * 8

    xP = jnp.concatenate([x[:, :H], x[:, H:]], axis=0)
    src = edge_index[0]
    dst = edge_index[1]
    srcs2 = jnp.concatenate([src, src + N]).reshape(2 * E // K, K)
    dst2 = dst.reshape(E // K, K)
    zA = jnp.zeros((rA, H), jnp.float32)
    zC = jnp.zeros((rA, 16), jnp.float32)
    onesK = jnp.ones((K, 16), jnp.float32)

    agg2, cnt = _sc_segment_sum(xP, srcs2, dst2, zA, zC, onesK,
                                N=N, E=E, H=H, K=K)

    WnT = W_neigh[:H, :]
    WnB = W_neigh[H:, :]
    b2 = b.reshape(1, D)

    self_out = _tc_self(x, W_self, b2, N=N, D=D, B=2000)
    return _tc_neigh(self_out, agg2, cnt, WnT, WnB, N=N, D=D, B=2000)


# speedup vs baseline: 7.5067x; 1.0253x over previous
import functools

import jax
import jax.numpy as jnp
from jax import lax
from jax.experimental import pallas as pl
from jax.experimental.pallas import tpu as pltpu
from jax.experimental.pallas import tpu_sc as plsc


def _sc_segment_sum(xP, srcs2, dst2, zA, zC, onesK, *, N, E, H, K):
    info = plsc.get_sparse_core_info()
    NC, NS = info.num_cores, info.num_subcores
    NB = E // K
    bpw = NB // NS
    nx = NB - bpw * NS
    rA = ((N + NS - 1) // NS + 7) // 8 * 8
    rB = N - rA * (NS - 1)
    assert bpw % 2 == 0 and rB > 0 and rB % 8 == 0

    mesh = plsc.VectorSubcoreMesh(core_axis_name="c", subcore_axis_name="s")

    @functools.partial(
        pl.kernel,
        out_type=(jax.ShapeDtypeStruct((NC * N, H), jnp.float32),
                  jax.ShapeDtypeStruct((N, 16), jnp.float32)),
        mesh=mesh,
        scratch_types=[
            pltpu.VMEM((bpw + 1, K), jnp.int32),
            pltpu.VMEM((bpw + 1, K), jnp.int32),
            pltpu.VMEM((K, H), jnp.float32),
            pltpu.VMEM((K, H), jnp.float32),
            pltpu.VMEM((K, 16), jnp.float32),
            pltpu.VMEM_SHARED((N, H), jnp.float32),
            pltpu.VMEM_SHARED((N, 16), jnp.float32),
            pltpu.SemaphoreType.DMA,
            pltpu.SemaphoreType.DMA,
            pltpu.SemaphoreType.DMA,
            pltpu.SemaphoreType.DMA,
            pltpu.SemaphoreType.DMA,
        ],
        compiler_params=pltpu.CompilerParams(use_tc_tiling_on_sc=False),
    )
    def sc_agg(xp_hbm, src_hbm, dst_hbm, za_hbm, zc_hbm, ones_hbm,
               out_hbm, cnt_hbm,
               si, di, rows0, rows1, onesv, acc, cnt,
               gsem0, gsem1, ssem0, ssem1, csem):
        c = lax.axis_index("c")
        s = lax.axis_index("s")

        @pl.when(s < NS - 1)
        def _():
            pltpu.sync_copy(za_hbm.at[pl.ds(0, rA)],
                            acc.at[pl.ds(s * rA, rA)])

        @pl.when(s == NS - 1)
        def _():
            pltpu.sync_copy(za_hbm.at[pl.ds(0, rB)],
                            acc.at[pl.ds((NS - 1) * rA, rB)])

        @pl.when((c == 0) & (s < NS - 1))
        def _():
            pltpu.sync_copy(zc_hbm.at[pl.ds(0, rA)],
                            cnt.at[pl.ds(s * rA, rA)])

        @pl.when((c == 0) & (s == NS - 1))
        def _():
            pltpu.sync_copy(zc_hbm.at[pl.ds(0, rB)],
                            cnt.at[pl.ds((NS - 1) * rA, rB)])

        @pl.when(c == 0)
        def _():
            pltpu.sync_copy(ones_hbm, onesv)

        b0 = s * bpw
        pltpu.sync_copy(src_hbm.at[pl.ds(c * NB + b0, bpw)],
                        si.at[pl.ds(0, bpw)])
        pltpu.sync_copy(dst_hbm.at[pl.ds(b0, bpw)], di.at[pl.ds(0, bpw)])

        @pl.when(s < nx)
        def _():
            xb = NB - nx + s
            pltpu.sync_copy(src_hbm.at[pl.ds(c * NB + xb, 1)],
                            si.at[pl.ds(bpw, 1)])
            pltpu.sync_copy(dst_hbm.at[pl.ds(xb, 1)], di.at[pl.ds(bpw, 1)])

        plsc.subcore_barrier()

        rows = (rows0, rows1)
        gsem = (gsem0, gsem1)
        ssem = (ssem0, ssem1)

        pltpu.async_copy(xp_hbm.at[si.at[0]], rows0, gsem0)

        def step(j, k):
            @pl.when(j >= 1)
            def _():
                pltpu.make_async_copy(
                    rows[1 - k], acc.at[di.at[0]], ssem[1 - k]).wait()

            @pl.when(j + 1 < bpw)
            def _():
                pltpu.async_copy(xp_hbm.at[si.at[j + 1]], rows[1 - k],
                                 gsem[1 - k])

            pltpu.make_async_copy(xp_hbm.at[si.at[0]], rows[k],
                                  gsem[k]).wait()
            pltpu.async_copy(rows[k], acc.at[di.at[j]], ssem[k], add=True)

            @pl.when((c == 0) & (j >= 1))
            def _():
                pltpu.make_async_copy(onesv, cnt.at[di.at[0]], csem).wait()

            @pl.when(c == 0)
            def _():
                pltpu.async_copy(onesv, cnt.at[di.at[j]], csem, add=True)

        def body(g, carry):
            step(2 * g, 0)
            step(2 * g + 1, 1)
            return carry

        lax.fori_loop(0, bpw // 2, body, 0, unroll=False)

        pltpu.make_async_copy(rows1, acc.at[di.at[0]], ssem1).wait()

        @pl.when(c == 0)
        def _():
            pltpu.make_async_copy(onesv, cnt.at[di.at[0]], csem).wait()

        @pl.when(s < nx)
        def _():
            pltpu.async_copy(xp_hbm.at[si.at[bpw]], rows0, gsem0).wait()
            pltpu.async_copy(rows0, acc.at[di.at[bpw]], ssem0,
                             add=True).wait()

            @pl.when(c == 0)
            def _():
                pltpu.async_copy(onesv, cnt.at[di.at[bpw]], csem,
                                 add=True).wait()

        plsc.subcore_barrier()

        @pl.when(s < NS - 1)
        def _():
            pltpu.sync_copy(acc.at[pl.ds(s * rA, rA)],
                            out_hbm.at[pl.ds(c * N + s * rA, rA)])

        @pl.when(s == NS - 1)
        def _():
            pltpu.sync_copy(acc.at[pl.ds((NS - 1) * rA, rB)],
                            out_hbm.at[pl.ds(c * N + (NS - 1) * rA, rB)])

        @pl.when((c == 0) & (s < NS - 1))
        def _():
            pltpu.sync_copy(cnt.at[pl.ds(s * rA, rA)],
                            cnt_hbm.at[pl.ds(s * rA, rA)])

        @pl.when((c == 0) & (s == NS - 1))
        def _():
            pltpu.sync_copy(cnt.at[pl.ds((NS - 1) * rA, rB)],
                            cnt_hbm.at[pl.ds((NS - 1) * rA, rB)])

    return sc_agg(xP, srcs2, dst2, zA, zC, onesK)


def _tc_dense(x, agg2, cnt, W_self, WnT, WnB, b2, *, N, D, B):
    H = D // 2

    def body(x_ref, al_ref, ar_ref, cnt_ref, ws_ref, wt_ref, wb_ref,
             b_ref, out_ref):
        r = 1.0 / jnp.maximum(cnt_ref[:, :1], 1.0)
        dn = (((1,), (0,)), ((), ()))
        acc = lax.dot_general(x_ref[...], ws_ref[...], dn,
                              precision=lax.Precision.HIGHEST,
                              preferred_element_type=jnp.float32)
        acc += lax.dot_general(al_ref[...] * r, wt_ref[...], dn,
                               precision=lax.Precision.HIGHEST,
                               preferred_element_type=jnp.float32)
        acc += lax.dot_general(ar_ref[...] * r, wb_ref[...], dn,
                               precision=lax.Precision.HIGHEST,
                               preferred_element_type=jnp.float32)
        out_ref[...] = acc + b_ref[...]

    nb = N // B
    return pl.pallas_call(
        body,
        grid=(nb,),
        in_specs=[
            pl.BlockSpec((B, D), lambda i: (i, 0)),
            pl.BlockSpec((B, H), lambda i: (i, 0)),
            pl.BlockSpec((B, H), lambda i: (nb + i, 0)),
            pl.BlockSpec((B, 16), lambda i: (i, 0)),
            pl.BlockSpec((D, D), lambda i: (0, 0)),
            pl.BlockSpec((H, D), lambda i: (0, 0)),
            pl.BlockSpec((H, D), lambda i: (0, 0)),
            pl.BlockSpec((1, D), lambda i: (0, 0)),
        ],
        out_specs=pl.BlockSpec((B, D), lambda i: (i, 0)),
        out_shape=jax.ShapeDtypeStruct((N, D), jnp.float32),
    )(x, agg2, agg2, cnt, W_self, WnT, WnB, b2)


def kernel(x, edge_index, W_self, W_neigh, b):
    N, D = x.shape
    E = edge_index.shape[1]
    H = D // 2
    K = 64
    NS = 16
    rA = ((N + NS - 1) // NS + 7) // 8 * 8

    xP = x.reshape(2 * N, H)
    src = edge_index[0]
    dst = edge_index[1]
    src2 = 2 * src
    srcs2 = jnp.concatenate([src2, src2 + 1]).reshape(2 * E // K, K)
    dst2 = dst.reshape(E // K, K)
    zA = jnp.zeros((rA, H), jnp.float32)
    zC = jnp.zeros((rA, 16), jnp.float32)
    onesK = jnp.ones((K, 16), jnp.float32)

    agg2, cnt = _sc_segment_sum(xP, srcs2, dst2, zA, zC, onesK,
                                N=N, E=E, H=H, K=K)

    WnT = W_neigh[:H, :]
    WnB = W_neigh[H:, :]
    b2 = b.reshape(1, D)

    return _tc_dense(x, agg2, cnt, W_self, WnT, WnB, b2, N=N, D=D, B=2000)
